# baseline (speedup 1.0000x reference)
import functools

import jax
import jax.numpy as jnp
from jax import lax
from jax.experimental import pallas as pl
from jax.experimental.pallas import tpu as pltpu
from jax.experimental.pallas import tpu_sc as plsc

N = 10000
E = 320000
BATCH_SIZE = 16
EPS = 1e-5

NC = 2
NS = 16
CHUNK = 128
EP = 327680
NR = 5056
ACC = 5120
DUMP = 5118
STRIPE = ACC // NS
KW = 128

_mesh = lambda: plsc.VectorSubcoreMesh(core_axis_name="c", subcore_axis_name="s")


def _fill(ref, rows, width, value):
    vec = jnp.full((16,), value, jnp.float32)

    def body(t, carry):
        for q in range(width // 16):
            ref[t, pl.ds(q * 16, 16)] = vec
        return carry

    lax.fori_loop(0, rows, body, 0)


def _make_scatter(feature_split):
    if feature_split:
        n_chunks = EP // (NS * CHUNK)
    else:
        n_chunks = EP // (NC * NS * CHUNK)

    @functools.partial(
        pl.kernel,
        mesh=_mesh(),
        out_type=jax.ShapeDtypeStruct((NC * ACC, KW), jnp.float32),
        scratch_types=[
            pltpu.VMEM((CHUNK,), jnp.int32),
            pltpu.VMEM((CHUNK,), jnp.int32),
            pltpu.VMEM((CHUNK, KW), jnp.float32),
            pltpu.VMEM((STRIPE, KW), jnp.float32),
            pltpu.VMEM_SHARED((ACC, KW), jnp.float32),
            pltpu.SemaphoreType.DMA,
        ],
    )
    def k(xs_hbm, src_hbm, dst_hbm, out_hbm, sidx, didx, rows, stage, acc, sem):
        c = lax.axis_index("c")
        s = lax.axis_index("s")
        _fill(stage, STRIPE, KW, 0.0)
        pltpu.sync_copy(stage, acc.at[pl.ds(s * STRIPE, STRIPE)])
        plsc.subcore_barrier()

        if feature_split:
            coff = jnp.full((16,), N, jnp.int32) * c
            wbase = s * (n_chunks * CHUNK)
        else:
            wbase = (c * NS + s) * (n_chunks * CHUNK)

        def body(i, carry):
            base = wbase + i * CHUNK
            pltpu.sync_copy(src_hbm.at[pl.ds(base, CHUNK)], sidx)
            pltpu.sync_copy(dst_hbm.at[pl.ds(base, CHUNK)], didx)
            if feature_split:
                for j in range(CHUNK // 16):
                    sidx[pl.ds(j * 16, 16)] = sidx[pl.ds(j * 16, 16)] + coff
            pltpu.async_copy(xs_hbm.at[sidx], rows, sem).wait()
            pltpu.sync_copy(rows, acc.at[didx], add=True)
            return carry

        lax.fori_loop(0, n_chunks, body, 0)
        plsc.subcore_barrier()
        pltpu.sync_copy(acc.at[pl.ds(s * STRIPE, STRIPE)], stage)
        pltpu.sync_copy(stage, out_hbm.at[pl.ds(c * ACC + s * STRIPE, STRIPE)])

    return k


_scatter_fsplit = _make_scatter(True)


def _head_kernel(pooled_ref, fc1W_ref, fc1b_ref, fc2W_ref, fc2b_ref, y_ref):
    p = pooled_ref[...]
    h = jnp.maximum(p @ fc1W_ref[...].T + fc1b_ref[...], 0.0)
    y_ref[...] = h @ fc2W_ref[...].T + fc2b_ref[...]


def kernel(x, edge_index, batch, W0, b0, g0, be0, W1, b1, g1, be1, W2, b2, g2, be2, W3, b3, g3, be3, fc1_W, fc1_b, fc2_W, fc2_b):
    Ws = [W0, W1, W2, W3]; bs = [b0, b1, b2, b3]; gs = [g0, g1, g2, g3]; bes = [be0, be1, be2, be3]
    pad = EP - E
    srcp = jnp.concatenate([edge_index[0], jnp.zeros((pad,), jnp.int32)])
    dstp = jnp.concatenate([edge_index[1], jnp.full((pad,), N, jnp.int32)])
    dloc = [jnp.where((dstp >= r * NR) & (dstp < r * NR + NR + (ACC - NR - 1)),
                      dstp - r * NR, DUMP).astype(jnp.int32) for r in range(2)]

    ones_cat = jnp.ones((2 * N, KW), jnp.float32)
    deg_parts = [_scatter_fsplit(ones_cat, srcp, dloc[r]) for r in range(2)]
    deg = jnp.concatenate([deg_parts[0][:NR, 0], deg_parts[1][:N - NR, 0]]) + 1.0
    dinv = lax.rsqrt(deg)

    h = x
    for i in range(4):
        K = Ws[i].shape[0]
        hw = h @ Ws[i].T
        xs = hw * dinv[:, None]
        if K == KW:
            xs_cat = jnp.concatenate([xs, xs], axis=0)
        else:
            xs_cat = jnp.concatenate([xs[:, :KW], xs[:, KW:]], axis=0)
        blocks = []
        for r in range(2):
            out = _scatter_fsplit(xs_cat, srcp, dloc[r])
            rows = NR if r == 0 else N - NR
            if K == KW:
                blocks.append(out[:rows])
            else:
                blocks.append(jnp.concatenate([out[:rows], out[ACC:ACC + rows]], axis=1))
        s = jnp.concatenate(blocks, axis=0)
        p = dinv[:, None] * (s + xs)
        h = jnp.maximum(p + bs[i], 0.0)
        mean = jnp.mean(h, axis=0)
        var = jnp.mean((h - mean) ** 2, axis=0)
        h = (h - mean) / jnp.sqrt(var + EPS) * gs[i] + bes[i]

    sm = jax.ops.segment_sum(h, batch, num_segments=BATCH_SIZE)
    cnt = jax.ops.segment_sum(jnp.ones((N,), dtype=h.dtype), batch, num_segments=BATCH_SIZE)
    pooled = sm / jnp.maximum(cnt, 1.0)[:, None]

    y = pl.pallas_call(
        _head_kernel,
        out_shape=jax.ShapeDtypeStruct((BATCH_SIZE, fc2_W.shape[0]), jnp.float32),
    )(pooled, fc1_W, fc1_b, fc2_W, fc2_b)
    return (y, pooled)
